# initial kernel scaffold (unmeasured)
import jax
import jax.numpy as jnp
from jax import lax
from jax.experimental import pallas as pl
from jax.experimental.pallas import tpu as pltpu

N_DEV = 8
M_PER = 512
N_OUT = 2048
N_HALF = N_OUT // 2
K_PER = 512


def kernel(x, w_mat):
    m_total, k_per = x.shape
    _, n_out = w_mat.shape
    assert m_total == N_DEV * M_PER and k_per == K_PER and n_out == N_OUT

    def body(x_ref, w_ref, out_ref,
             comm_up, comm_dn, maxbuf,
             send_up, recv_up, send_dn, recv_dn,
             bfly_send, bfly_recv):
        my = lax.axis_index("i")
        right = lax.rem(my + 1, N_DEV)
        left = lax.rem(my + N_DEV - 1, N_DEV)

        barrier_sem = pltpu.get_barrier_semaphore()
        for nbr in (left, right):
            pl.semaphore_signal(
                barrier_sem, inc=1,
                device_id=(nbr,), device_id_type=pl.DeviceIdType.MESH,
            )
        pl.semaphore_wait(barrier_sem, 2)

        def partial_chunk(c, lo):
            xc = x_ref[pl.ds(c * M_PER, M_PER), :]
            return lax.dot_general(
                xc, w_ref[:, lo:lo + N_HALF],
                (((1,), (0,)), ((), ())),
                precision=lax.Precision.HIGHEST,
                preferred_element_type=jnp.float32,
            )

        for s in range(N_DEV - 1):
            c_up = lax.rem(my + (N_DEV - 1 - s) * 1 + N_DEV, N_DEV)
            c_dn = lax.rem(my + 1 + s, N_DEV)
            p_up = partial_chunk(c_up, 0)
            p_dn = partial_chunk(c_dn, N_HALF)
            if s == 0:
                comm_up[0] = p_up
                comm_dn[0] = p_dn
            else:
                comm_up[s] = comm_up[s] + p_up
                comm_dn[s] = comm_dn[s] + p_dn
            rdma_up = pltpu.make_async_remote_copy(
                src_ref=comm_up.at[s], dst_ref=comm_up.at[s + 1],
                send_sem=send_up.at[s], recv_sem=recv_up.at[s],
                device_id=(right,), device_id_type=pl.DeviceIdType.MESH,
            )
            rdma_dn = pltpu.make_async_remote_copy(
                src_ref=comm_dn.at[s], dst_ref=comm_dn.at[s + 1],
                send_sem=send_dn.at[s], recv_sem=recv_dn.at[s],
                device_id=(left,), device_id_type=pl.DeviceIdType.MESH,
            )
            rdma_up.start()
            rdma_dn.start()
            rdma_up.wait()
            rdma_dn.wait()

        y_l = jnp.maximum(comm_up[N_DEV - 1] + partial_chunk(my, 0), 0.0)
        y_r = jnp.maximum(
            comm_dn[N_DEV - 1] + partial_chunk(my, N_HALF), 0.0)
        out_ref[:, 0:N_HALF] = y_l
        out_ref[:, N_HALF:N_OUT] = y_r

        local_max = jnp.maximum(jnp.max(y_l), jnp.max(y_r))
        maxbuf[3] = jnp.full((8, 128), local_max, jnp.float32)
        for r, d in enumerate((1, 2, 4)):
            partner = jnp.bitwise_xor(my, d)
            ex = pltpu.make_async_remote_copy(
                src_ref=maxbuf.at[3], dst_ref=maxbuf.at[r],
                send_sem=bfly_send.at[r], recv_sem=bfly_recv.at[r],
                device_id=(partner,), device_id_type=pl.DeviceIdType.MESH,
            )
            ex.start()
            ex.wait()
            maxbuf[3] = jnp.maximum(maxbuf[3], maxbuf[r])

        amax = jnp.max(maxbuf[3])
        scale = amax / 448.0
        inv = jnp.where(scale > 0.0, 1.0 / scale, 0.0)
        q = (out_ref[:, :] * inv).astype(jnp.float8_e4m3fn)
        out_ref[:, :] = q.astype(jnp.float32) * scale

    return pl.pallas_call(
        body,
        out_shape=jax.ShapeDtypeStruct((M_PER, N_OUT), jnp.float32),
        in_specs=[
            pl.BlockSpec(memory_space=pltpu.VMEM),
            pl.BlockSpec(memory_space=pltpu.VMEM),
        ],
        out_specs=pl.BlockSpec(memory_space=pltpu.VMEM),
        scratch_shapes=[
            pltpu.VMEM((N_DEV, M_PER, N_HALF), jnp.float32),
            pltpu.VMEM((N_DEV, M_PER, N_HALF), jnp.float32),
            pltpu.VMEM((4, 8, 128), jnp.float32),
            pltpu.SemaphoreType.DMA((N_DEV - 1,)),
            pltpu.SemaphoreType.DMA((N_DEV - 1,)),
            pltpu.SemaphoreType.DMA((N_DEV - 1,)),
            pltpu.SemaphoreType.DMA((N_DEV - 1,)),
            pltpu.SemaphoreType.DMA((3,)),
            pltpu.SemaphoreType.DMA((3,)),
        ],
        compiler_params=pltpu.CompilerParams(collective_id=0),
    )(x, w_mat)


# baseline (device time: 255261 ns/iter reference)
import jax
import jax.numpy as jnp
from jax import lax
from jax.experimental import pallas as pl
from jax.experimental.pallas import tpu as pltpu

N_DEV = 8
M_PER = 512
N_OUT = 2048
N_HALF = N_OUT // 2
K_PER = 512


def kernel(x, w_mat):
    m_total, k_per = x.shape
    _, n_out = w_mat.shape
    assert m_total == N_DEV * M_PER and k_per == K_PER and n_out == N_OUT

    def body(x_ref, w_ref, out_ref,
             comm_up, comm_dn, maxbuf,
             send_up, recv_up, send_dn, recv_dn,
             bfly_send, bfly_recv):
        my = lax.axis_index("i")
        right = lax.rem(my + 1, N_DEV)
        left = lax.rem(my + N_DEV - 1, N_DEV)

        barrier_sem = pltpu.get_barrier_semaphore()
        for nbr in (left, right):
            pl.semaphore_signal(
                barrier_sem, inc=1,
                device_id=(nbr,), device_id_type=pl.DeviceIdType.MESH,
            )
        pl.semaphore_wait(barrier_sem, 2)

        def partial_chunk(c, lo):
            xc = x_ref[pl.ds(c * M_PER, M_PER), :]
            return lax.dot_general(
                xc, w_ref[:, lo:lo + N_HALF],
                (((1,), (0,)), ((), ())),
                precision=lax.Precision.HIGHEST,
                preferred_element_type=jnp.float32,
            )

        for s in range(N_DEV - 1):
            c_up = lax.rem(my + N_DEV - 1 - s, N_DEV)
            c_dn = lax.rem(my + 1 + s, N_DEV)
            p_up = partial_chunk(c_up, 0)
            p_dn = partial_chunk(c_dn, N_HALF)
            if s == 0:
                comm_up[0, :, :] = p_up
                comm_dn[0, :, :] = p_dn
            else:
                comm_up[s, :, :] = comm_up[s, :, :] + p_up
                comm_dn[s, :, :] = comm_dn[s, :, :] + p_dn
            rdma_up = pltpu.make_async_remote_copy(
                src_ref=comm_up.at[s], dst_ref=comm_up.at[s + 1],
                send_sem=send_up.at[s], recv_sem=recv_up.at[s],
                device_id=(right,), device_id_type=pl.DeviceIdType.MESH,
            )
            rdma_dn = pltpu.make_async_remote_copy(
                src_ref=comm_dn.at[s], dst_ref=comm_dn.at[s + 1],
                send_sem=send_dn.at[s], recv_sem=recv_dn.at[s],
                device_id=(left,), device_id_type=pl.DeviceIdType.MESH,
            )
            rdma_up.start()
            rdma_dn.start()
            rdma_up.wait()
            rdma_dn.wait()

        y_l = jnp.maximum(
            comm_up[N_DEV - 1, :, :] + partial_chunk(my, 0), 0.0)
        y_r = jnp.maximum(
            comm_dn[N_DEV - 1, :, :] + partial_chunk(my, N_HALF), 0.0)
        out_ref[:, 0:N_HALF] = y_l
        out_ref[:, N_HALF:N_OUT] = y_r

        local_max = jnp.maximum(jnp.max(y_l), jnp.max(y_r))
        maxbuf[3, :, :] = jnp.full((8, 128), local_max, jnp.float32)
        for r, d in enumerate((1, 2, 4)):
            partner = jnp.bitwise_xor(my, d)
            ex = pltpu.make_async_remote_copy(
                src_ref=maxbuf.at[3], dst_ref=maxbuf.at[r],
                send_sem=bfly_send.at[r], recv_sem=bfly_recv.at[r],
                device_id=(partner,), device_id_type=pl.DeviceIdType.MESH,
            )
            ex.start()
            ex.wait()
            maxbuf[3, :, :] = jnp.maximum(maxbuf[3, :, :], maxbuf[r, :, :])

        amax = jnp.max(maxbuf[3, :, :])
        scale = amax / 448.0
        inv = jnp.where(scale > 0.0, 1.0 / scale, 0.0)
        q = (out_ref[:, :] * inv).astype(jnp.float8_e4m3fn)
        out_ref[:, :] = q.astype(jnp.float32) * scale

    return pl.pallas_call(
        body,
        out_shape=jax.ShapeDtypeStruct((M_PER, N_OUT), jnp.float32),
        in_specs=[
            pl.BlockSpec(memory_space=pltpu.VMEM),
            pl.BlockSpec(memory_space=pltpu.VMEM),
        ],
        out_specs=pl.BlockSpec(memory_space=pltpu.VMEM),
        scratch_shapes=[
            pltpu.VMEM((N_DEV, M_PER, N_HALF), jnp.float32),
            pltpu.VMEM((N_DEV, M_PER, N_HALF), jnp.float32),
            pltpu.VMEM((4, 8, 128), jnp.float32),
            pltpu.SemaphoreType.DMA((N_DEV - 1,)),
            pltpu.SemaphoreType.DMA((N_DEV - 1,)),
            pltpu.SemaphoreType.DMA((N_DEV - 1,)),
            pltpu.SemaphoreType.DMA((N_DEV - 1,)),
            pltpu.SemaphoreType.DMA((3,)),
            pltpu.SemaphoreType.DMA((3,)),
        ],
        compiler_params=pltpu.CompilerParams(
            collective_id=0, vmem_limit_bytes=100 * 1024 * 1024),
    )(x, w_mat)


# device time: 205641 ns/iter; 1.2413x vs baseline; 1.2413x over previous
import jax
import jax.numpy as jnp
from jax import lax
from jax.experimental import pallas as pl
from jax.experimental.pallas import tpu as pltpu

N_DEV = 8
M_PER = 512
N_OUT = 2048
N_HALF = N_OUT // 2
K_PER = 512


def kernel(x, w_mat):
    m_total, k_per = x.shape
    _, n_out = w_mat.shape
    assert m_total == N_DEV * M_PER and k_per == K_PER and n_out == N_OUT

    def body(x_ref, w_ref, out_ref,
             comm_up, comm_dn, maxbuf,
             send_up, recv_up, send_dn, recv_dn,
             bfly_send, bfly_recv):
        my = lax.axis_index("i")
        right = lax.rem(my + 1, N_DEV)
        left = lax.rem(my + N_DEV - 1, N_DEV)

        barrier_sem = pltpu.get_barrier_semaphore()
        for nbr in (left, right):
            pl.semaphore_signal(
                barrier_sem, inc=1,
                device_id=(nbr,), device_id_type=pl.DeviceIdType.MESH,
            )
        pl.semaphore_wait(barrier_sem, 2)

        def partial_chunk(c, lo):
            xc = x_ref[pl.ds(c * M_PER, M_PER), :]
            return lax.dot_general(
                xc, w_ref[:, lo:lo + N_HALF],
                (((1,), (0,)), ((), ())),
                precision=lax.Precision.HIGHEST,
                preferred_element_type=jnp.float32,
            )

        def make_hop(s):
            rdma_up = pltpu.make_async_remote_copy(
                src_ref=comm_up.at[s], dst_ref=comm_up.at[s + 1],
                send_sem=send_up.at[s], recv_sem=recv_up.at[s],
                device_id=(right,), device_id_type=pl.DeviceIdType.MESH,
            )
            rdma_dn = pltpu.make_async_remote_copy(
                src_ref=comm_dn.at[s], dst_ref=comm_dn.at[s + 1],
                send_sem=send_dn.at[s], recv_sem=recv_dn.at[s],
                device_id=(left,), device_id_type=pl.DeviceIdType.MESH,
            )
            return rdma_up, rdma_dn

        hops = []
        comm_up[0, :, :] = partial_chunk(lax.rem(my + N_DEV - 1, N_DEV), 0)
        comm_dn[0, :, :] = partial_chunk(lax.rem(my + 1, N_DEV), N_HALF)
        rdma_up, rdma_dn = make_hop(0)
        rdma_up.start()
        rdma_dn.start()
        hops.append((rdma_up, rdma_dn))

        for s in range(1, N_DEV - 1):
            c_up = lax.rem(my + N_DEV - 1 - s, N_DEV)
            c_dn = lax.rem(my + 1 + s, N_DEV)
            p_up = partial_chunk(c_up, 0)
            p_dn = partial_chunk(c_dn, N_HALF)
            hops[s - 1][0].wait_recv()
            hops[s - 1][1].wait_recv()
            comm_up[s, :, :] = comm_up[s, :, :] + p_up
            comm_dn[s, :, :] = comm_dn[s, :, :] + p_dn
            rdma_up, rdma_dn = make_hop(s)
            rdma_up.start()
            rdma_dn.start()
            hops.append((rdma_up, rdma_dn))

        p_l = partial_chunk(my, 0)
        p_r = partial_chunk(my, N_HALF)
        hops[N_DEV - 2][0].wait_recv()
        hops[N_DEV - 2][1].wait_recv()
        y_l = jnp.maximum(comm_up[N_DEV - 1, :, :] + p_l, 0.0)
        y_r = jnp.maximum(comm_dn[N_DEV - 1, :, :] + p_r, 0.0)
        out_ref[:, 0:N_HALF] = y_l
        out_ref[:, N_HALF:N_OUT] = y_r
        for rdma_up, rdma_dn in hops:
            rdma_up.wait_send()
            rdma_dn.wait_send()

        local_max = jnp.maximum(jnp.max(y_l), jnp.max(y_r))
        maxbuf[3, :, :] = jnp.full((8, 128), local_max, jnp.float32)
        for r, d in enumerate((1, 2, 4)):
            partner = jnp.bitwise_xor(my, d)
            ex = pltpu.make_async_remote_copy(
                src_ref=maxbuf.at[3], dst_ref=maxbuf.at[r],
                send_sem=bfly_send.at[r], recv_sem=bfly_recv.at[r],
                device_id=(partner,), device_id_type=pl.DeviceIdType.MESH,
            )
            ex.start()
            ex.wait()
            maxbuf[3, :, :] = jnp.maximum(maxbuf[3, :, :], maxbuf[r, :, :])

        amax = jnp.max(maxbuf[3, :, :])
        scale = amax / 448.0
        inv = jnp.where(scale > 0.0, 1.0 / scale, 0.0)
        q = (out_ref[:, :] * inv).astype(jnp.float8_e4m3fn)
        out_ref[:, :] = q.astype(jnp.float32) * scale

    return pl.pallas_call(
        body,
        out_shape=jax.ShapeDtypeStruct((M_PER, N_OUT), jnp.float32),
        in_specs=[
            pl.BlockSpec(memory_space=pltpu.VMEM),
            pl.BlockSpec(memory_space=pltpu.VMEM),
        ],
        out_specs=pl.BlockSpec(memory_space=pltpu.VMEM),
        scratch_shapes=[
            pltpu.VMEM((N_DEV, M_PER, N_HALF), jnp.float32),
            pltpu.VMEM((N_DEV, M_PER, N_HALF), jnp.float32),
            pltpu.VMEM((4, 8, 128), jnp.float32),
            pltpu.SemaphoreType.DMA((N_DEV - 1,)),
            pltpu.SemaphoreType.DMA((N_DEV - 1,)),
            pltpu.SemaphoreType.DMA((N_DEV - 1,)),
            pltpu.SemaphoreType.DMA((N_DEV - 1,)),
            pltpu.SemaphoreType.DMA((3,)),
            pltpu.SemaphoreType.DMA((3,)),
        ],
        compiler_params=pltpu.CompilerParams(
            collective_id=0, vmem_limit_bytes=100 * 1024 * 1024),
    )(x, w_mat)


# device time: 176410 ns/iter; 1.4470x vs baseline; 1.1657x over previous
import jax
import jax.numpy as jnp
from jax import lax
from jax.experimental import pallas as pl
from jax.experimental.pallas import tpu as pltpu

N_DEV = 8
M_PER = 512
N_OUT = 2048
N_HALF = N_OUT // 2
K_PER = 512


def kernel(x, w_mat):
    m_total, k_per = x.shape
    _, n_out = w_mat.shape
    assert m_total == N_DEV * M_PER and k_per == K_PER and n_out == N_OUT

    def body(x_ref, w_ref, out_ref,
             comm_up, comm_dn, res_up, res_dn, maxbuf,
             send_up, recv_up, send_dn, recv_dn,
             rsend_up, rrecv_up, rsend_dn, rrecv_dn,
             bfly_send, bfly_recv):
        my = lax.axis_index("i")
        right = lax.rem(my + 1, N_DEV)
        left = lax.rem(my + N_DEV - 1, N_DEV)

        barrier_sem = pltpu.get_barrier_semaphore()
        for nbr in (left, right):
            pl.semaphore_signal(
                barrier_sem, inc=1,
                device_id=(nbr,), device_id_type=pl.DeviceIdType.MESH,
            )
        pl.semaphore_wait(barrier_sem, 2)

        def partial_chunk(c, lo):
            xc = x_ref[pl.ds(c * M_PER, M_PER), :]
            return lax.dot_general(
                xc, w_ref[:, lo:lo + N_HALF],
                (((1,), (0,)), ((), ())),
                precision=lax.Precision.HIGHEST,
                preferred_element_type=jnp.float32,
            )

        def split_store(a, s, comm, res):
            b = a.astype(jnp.bfloat16)
            comm[s, :, :] = b
            res[s, :, :] = (a - b.astype(jnp.float32)).astype(
                jnp.float8_e5m2)

        def merge(s, comm, res):
            return (comm[s, :, :].astype(jnp.float32)
                    + res[s, :, :].astype(jnp.float32))

        def make_hop(s):
            rdmas = []
            for comm, res, sems, tgt in (
                (comm_up, res_up, (send_up, recv_up, rsend_up, rrecv_up),
                 right),
                (comm_dn, res_dn, (send_dn, recv_dn, rsend_dn, rrecv_dn),
                 left),
            ):
                rdmas.append(pltpu.make_async_remote_copy(
                    src_ref=comm.at[s], dst_ref=comm.at[s + 1],
                    send_sem=sems[0].at[s], recv_sem=sems[1].at[s],
                    device_id=(tgt,), device_id_type=pl.DeviceIdType.MESH,
                ))
                rdmas.append(pltpu.make_async_remote_copy(
                    src_ref=res.at[s], dst_ref=res.at[s + 1],
                    send_sem=sems[2].at[s], recv_sem=sems[3].at[s],
                    device_id=(tgt,), device_id_type=pl.DeviceIdType.MESH,
                ))
            return rdmas

        hops = []
        split_store(partial_chunk(lax.rem(my + N_DEV - 1, N_DEV), 0),
                    0, comm_up, res_up)
        split_store(partial_chunk(lax.rem(my + 1, N_DEV), N_HALF),
                    0, comm_dn, res_dn)
        hop = make_hop(0)
        for r_ in hop:
            r_.start()
        hops.append(hop)

        for s in range(1, N_DEV - 1):
            c_up = lax.rem(my + N_DEV - 1 - s, N_DEV)
            c_dn = lax.rem(my + 1 + s, N_DEV)
            p_up = partial_chunk(c_up, 0)
            p_dn = partial_chunk(c_dn, N_HALF)
            for r_ in hops[s - 1]:
                r_.wait_recv()
            split_store(merge(s, comm_up, res_up) + p_up,
                        s, comm_up, res_up)
            split_store(merge(s, comm_dn, res_dn) + p_dn,
                        s, comm_dn, res_dn)
            hop = make_hop(s)
            for r_ in hop:
                r_.start()
            hops.append(hop)

        p_l = partial_chunk(my, 0)
        p_r = partial_chunk(my, N_HALF)
        for r_ in hops[N_DEV - 2]:
            r_.wait_recv()
        y_l = jnp.maximum(merge(N_DEV - 1, comm_up, res_up) + p_l, 0.0)
        y_r = jnp.maximum(merge(N_DEV - 1, comm_dn, res_dn) + p_r, 0.0)
        out_ref[:, 0:N_HALF] = y_l
        out_ref[:, N_HALF:N_OUT] = y_r
        for hop in hops:
            for r_ in hop:
                r_.wait_send()

        local_max = jnp.maximum(jnp.max(y_l), jnp.max(y_r))
        maxbuf[3, :, :] = jnp.full((8, 128), local_max, jnp.float32)
        for r, d in enumerate((1, 2, 4)):
            partner = jnp.bitwise_xor(my, d)
            ex = pltpu.make_async_remote_copy(
                src_ref=maxbuf.at[3], dst_ref=maxbuf.at[r],
                send_sem=bfly_send.at[r], recv_sem=bfly_recv.at[r],
                device_id=(partner,), device_id_type=pl.DeviceIdType.MESH,
            )
            ex.start()
            ex.wait()
            maxbuf[3, :, :] = jnp.maximum(maxbuf[3, :, :], maxbuf[r, :, :])

        amax = jnp.max(maxbuf[3, :, :])
        scale = amax / 448.0
        inv = jnp.where(scale > 0.0, 1.0 / scale, 0.0)
        q = (out_ref[:, :] * inv).astype(jnp.float8_e4m3fn)
        out_ref[:, :] = q.astype(jnp.float32) * scale

    return pl.pallas_call(
        body,
        out_shape=jax.ShapeDtypeStruct((M_PER, N_OUT), jnp.float32),
        in_specs=[
            pl.BlockSpec(memory_space=pltpu.VMEM),
            pl.BlockSpec(memory_space=pltpu.VMEM),
        ],
        out_specs=pl.BlockSpec(memory_space=pltpu.VMEM),
        scratch_shapes=[
            pltpu.VMEM((N_DEV, M_PER, N_HALF), jnp.bfloat16),
            pltpu.VMEM((N_DEV, M_PER, N_HALF), jnp.bfloat16),
            pltpu.VMEM((N_DEV, M_PER, N_HALF), jnp.float8_e5m2),
            pltpu.VMEM((N_DEV, M_PER, N_HALF), jnp.float8_e5m2),
            pltpu.VMEM((4, 8, 128), jnp.float32),
            pltpu.SemaphoreType.DMA((N_DEV - 1,)),
            pltpu.SemaphoreType.DMA((N_DEV - 1,)),
            pltpu.SemaphoreType.DMA((N_DEV - 1,)),
            pltpu.SemaphoreType.DMA((N_DEV - 1,)),
            pltpu.SemaphoreType.DMA((N_DEV - 1,)),
            pltpu.SemaphoreType.DMA((N_DEV - 1,)),
            pltpu.SemaphoreType.DMA((N_DEV - 1,)),
            pltpu.SemaphoreType.DMA((N_DEV - 1,)),
            pltpu.SemaphoreType.DMA((3,)),
            pltpu.SemaphoreType.DMA((3,)),
        ],
        compiler_params=pltpu.CompilerParams(
            collective_id=0, vmem_limit_bytes=100 * 1024 * 1024),
    )(x, w_mat)


# device time: 144378 ns/iter; 1.7680x vs baseline; 1.2219x over previous
import jax
import jax.numpy as jnp
from jax import lax
from jax.experimental import pallas as pl
from jax.experimental.pallas import tpu as pltpu

N_DEV = 8
M_PER = 512
N_OUT = 2048
N_HALF = N_OUT // 2
K_PER = 512


def kernel(x, w_mat):
    m_total, k_per = x.shape
    _, n_out = w_mat.shape
    assert m_total == N_DEV * M_PER and k_per == K_PER and n_out == N_OUT

    M_SUB = M_PER // 2

    def body(x_ref, w_ref, out_ref,
             comm_up, comm_dn, res_up, res_dn, maxbuf,
             send_up, recv_up, send_dn, recv_dn,
             rsend_up, rrecv_up, rsend_dn, rrecv_dn,
             a2a_send, a2a_recv):
        my = lax.axis_index("i")
        right = lax.rem(my + 1, N_DEV)
        left = lax.rem(my + N_DEV - 1, N_DEV)

        barrier_sem = pltpu.get_barrier_semaphore()
        for nbr in (left, right):
            pl.semaphore_signal(
                barrier_sem, inc=1,
                device_id=(nbr,), device_id_type=pl.DeviceIdType.MESH,
            )
        pl.semaphore_wait(barrier_sem, 2)

        def partial_rows(c, lo, j):
            xc = x_ref[pl.ds(c * M_PER + j * M_SUB, M_SUB), :]
            return lax.dot_general(
                xc, w_ref[:, lo:lo + N_HALF],
                (((1,), (0,)), ((), ())),
                precision=lax.Precision.HIGHEST,
                preferred_element_type=jnp.float32,
            )

        def rows(j):
            return pl.ds(j * M_SUB, M_SUB)

        def split_store(a, s, j, comm, res):
            b = a.astype(jnp.bfloat16)
            comm[s, rows(j), :] = b
            res[s, rows(j), :] = (a - b.astype(jnp.float32)).astype(
                jnp.float8_e5m2)

        def merge(s, j, comm, res):
            return (comm[s, rows(j), :].astype(jnp.float32)
                    + res[s, rows(j), :].astype(jnp.float32))

        def make_hop(s, j):
            rdmas = []
            for comm, res, sems, tgt in (
                (comm_up, res_up, (send_up, recv_up, rsend_up, rrecv_up),
                 right),
                (comm_dn, res_dn, (send_dn, recv_dn, rsend_dn, rrecv_dn),
                 left),
            ):
                rdmas.append(pltpu.make_async_remote_copy(
                    src_ref=comm.at[s, rows(j), :],
                    dst_ref=comm.at[s + 1, rows(j), :],
                    send_sem=sems[0].at[s, j], recv_sem=sems[1].at[s, j],
                    device_id=(tgt,), device_id_type=pl.DeviceIdType.MESH,
                ))
                rdmas.append(pltpu.make_async_remote_copy(
                    src_ref=res.at[s, rows(j), :],
                    dst_ref=res.at[s + 1, rows(j), :],
                    send_sem=sems[2].at[s, j], recv_sem=sems[3].at[s, j],
                    device_id=(tgt,), device_id_type=pl.DeviceIdType.MESH,
                ))
            return rdmas

        hops = []
        c_up0 = lax.rem(my + N_DEV - 1, N_DEV)
        c_dn0 = lax.rem(my + 1, N_DEV)
        subs = []
        for j in (0, 1):
            split_store(partial_rows(c_up0, 0, j), 0, j, comm_up, res_up)
            split_store(partial_rows(c_dn0, N_HALF, j), 0, j,
                        comm_dn, res_dn)
            sub = make_hop(0, j)
            for r_ in sub:
                r_.start()
            subs.append(sub)
        hops.append(subs)

        for s in range(1, N_DEV - 1):
            c_up = lax.rem(my + N_DEV - 1 - s, N_DEV)
            c_dn = lax.rem(my + 1 + s, N_DEV)
            p_up = [partial_rows(c_up, 0, j) for j in (0, 1)]
            p_dn = [partial_rows(c_dn, N_HALF, j) for j in (0, 1)]
            subs = []
            for j in (0, 1):
                for r_ in hops[s - 1][j]:
                    r_.wait_recv()
                split_store(merge(s, j, comm_up, res_up) + p_up[j],
                            s, j, comm_up, res_up)
                split_store(merge(s, j, comm_dn, res_dn) + p_dn[j],
                            s, j, comm_dn, res_dn)
                sub = make_hop(s, j)
                for r_ in sub:
                    r_.start()
                subs.append(sub)
            hops.append(subs)

        p_l = [partial_rows(my, 0, j) for j in (0, 1)]
        p_r = [partial_rows(my, N_HALF, j) for j in (0, 1)]
        local_max = jnp.float32(0.0)
        for j in (0, 1):
            for r_ in hops[N_DEV - 2][j]:
                r_.wait_recv()
            y_l = jnp.maximum(
                merge(N_DEV - 1, j, comm_up, res_up) + p_l[j], 0.0)
            y_r = jnp.maximum(
                merge(N_DEV - 1, j, comm_dn, res_dn) + p_r[j], 0.0)
            out_ref[rows(j), 0:N_HALF] = y_l
            out_ref[rows(j), N_HALF:N_OUT] = y_r
            local_max = jnp.maximum(
                local_max, jnp.maximum(jnp.max(y_l), jnp.max(y_r)))

        maxbuf[7, :, :] = jnp.full((8, 128), local_max, jnp.float32)
        a2a = []
        for d in range(1, N_DEV):
            tgt = lax.rem(my + d, N_DEV)
            o = N_DEV - 1 - d
            ex = pltpu.make_async_remote_copy(
                src_ref=maxbuf.at[7], dst_ref=maxbuf.at[o],
                send_sem=a2a_send.at[o], recv_sem=a2a_recv.at[o],
                device_id=(tgt,), device_id_type=pl.DeviceIdType.MESH,
            )
            ex.start()
            a2a.append(ex)
        for hop in hops:
            for sub in hop:
                for r_ in sub:
                    r_.wait_send()
        for ex in a2a:
            ex.wait_recv()
        for ex in a2a:
            ex.wait_send()

        amax = jnp.max(maxbuf[:, :, :])
        scale = amax / 448.0
        inv = jnp.where(scale > 0.0, 1.0 / scale, 0.0)
        q = (out_ref[:, :] * inv).astype(jnp.float8_e4m3fn)
        out_ref[:, :] = q.astype(jnp.float32) * scale

    return pl.pallas_call(
        body,
        out_shape=jax.ShapeDtypeStruct((M_PER, N_OUT), jnp.float32),
        in_specs=[
            pl.BlockSpec(memory_space=pltpu.VMEM),
            pl.BlockSpec(memory_space=pltpu.VMEM),
        ],
        out_specs=pl.BlockSpec(memory_space=pltpu.VMEM),
        scratch_shapes=[
            pltpu.VMEM((N_DEV, M_PER, N_HALF), jnp.bfloat16),
            pltpu.VMEM((N_DEV, M_PER, N_HALF), jnp.bfloat16),
            pltpu.VMEM((N_DEV, M_PER, N_HALF), jnp.float8_e5m2),
            pltpu.VMEM((N_DEV, M_PER, N_HALF), jnp.float8_e5m2),
            pltpu.VMEM((N_DEV, 8, 128), jnp.float32),
            pltpu.SemaphoreType.DMA((N_DEV - 1, 2)),
            pltpu.SemaphoreType.DMA((N_DEV - 1, 2)),
            pltpu.SemaphoreType.DMA((N_DEV - 1, 2)),
            pltpu.SemaphoreType.DMA((N_DEV - 1, 2)),
            pltpu.SemaphoreType.DMA((N_DEV - 1, 2)),
            pltpu.SemaphoreType.DMA((N_DEV - 1, 2)),
            pltpu.SemaphoreType.DMA((N_DEV - 1, 2)),
            pltpu.SemaphoreType.DMA((N_DEV - 1, 2)),
            pltpu.SemaphoreType.DMA((N_DEV - 1,)),
            pltpu.SemaphoreType.DMA((N_DEV - 1,)),
        ],
        compiler_params=pltpu.CompilerParams(
            collective_id=0, vmem_limit_bytes=100 * 1024 * 1024),
    )(x, w_mat)


# device time: 128291 ns/iter; 1.9897x vs baseline; 1.1254x over previous
import jax
import jax.numpy as jnp
from jax import lax
from jax.experimental import pallas as pl
from jax.experimental.pallas import tpu as pltpu

N_DEV = 8
M_PER = 512
N_OUT = 2048
N_HALF = N_OUT // 2
K_PER = 512


def kernel(x, w_mat):
    m_total, k_per = x.shape
    _, n_out = w_mat.shape
    assert m_total == N_DEV * M_PER and k_per == K_PER and n_out == N_OUT

    M_SUB = M_PER // 2

    def body(x_ref, w_ref, out_ref,
             comm_up, comm_dn, res_up, res_dn, maxbuf,
             send_up, recv_up, send_dn, recv_dn,
             rsend_up, rrecv_up, rsend_dn, rrecv_dn,
             a2a_send, a2a_recv):
        my = lax.axis_index("i")
        right = lax.rem(my + 1, N_DEV)
        left = lax.rem(my + N_DEV - 1, N_DEV)

        barrier_sem = pltpu.get_barrier_semaphore()
        for nbr in (left, right):
            pl.semaphore_signal(
                barrier_sem, inc=1,
                device_id=(nbr,), device_id_type=pl.DeviceIdType.MESH,
            )
        pl.semaphore_wait(barrier_sem, 2)

        def partial_rows(c, lo, j):
            xc = x_ref[pl.ds(c * M_PER + j * M_SUB, M_SUB), :]
            return lax.dot_general(
                xc, w_ref[:, lo:lo + N_HALF],
                (((1,), (0,)), ((), ())),
                precision=lax.Precision.HIGHEST,
                preferred_element_type=jnp.float32,
            )

        RES_FROM = 3

        def rows(j):
            return pl.ds(j * M_SUB, M_SUB)

        def split_store(a, s, j, comm, res):
            b = a.astype(jnp.bfloat16)
            comm[s, rows(j), :] = b
            if s >= RES_FROM:
                res[s, rows(j), :] = (a - b.astype(jnp.float32)).astype(
                    jnp.float8_e5m2)

        def merge(s, j, comm, res):
            m = comm[s, rows(j), :].astype(jnp.float32)
            if s >= RES_FROM + 1:
                m = m + res[s, rows(j), :].astype(jnp.float32)
            return m

        def make_hop(s, j):
            rdmas = []
            for comm, res, sems, tgt in (
                (comm_up, res_up, (send_up, recv_up, rsend_up, rrecv_up),
                 right),
                (comm_dn, res_dn, (send_dn, recv_dn, rsend_dn, rrecv_dn),
                 left),
            ):
                rdmas.append(pltpu.make_async_remote_copy(
                    src_ref=comm.at[s, rows(j), :],
                    dst_ref=comm.at[s + 1, rows(j), :],
                    send_sem=sems[0].at[s, j], recv_sem=sems[1].at[s, j],
                    device_id=(tgt,), device_id_type=pl.DeviceIdType.MESH,
                ))
                if s >= RES_FROM:
                    rdmas.append(pltpu.make_async_remote_copy(
                        src_ref=res.at[s, rows(j), :],
                        dst_ref=res.at[s + 1, rows(j), :],
                        send_sem=sems[2].at[s, j],
                        recv_sem=sems[3].at[s, j],
                        device_id=(tgt,),
                        device_id_type=pl.DeviceIdType.MESH,
                    ))
            return rdmas

        hops = []
        c_up0 = lax.rem(my + N_DEV - 1, N_DEV)
        c_dn0 = lax.rem(my + 1, N_DEV)
        subs = []
        for j in (0, 1):
            split_store(partial_rows(c_up0, 0, j), 0, j, comm_up, res_up)
            split_store(partial_rows(c_dn0, N_HALF, j), 0, j,
                        comm_dn, res_dn)
            sub = make_hop(0, j)
            for r_ in sub:
                r_.start()
            subs.append(sub)
        hops.append(subs)

        for s in range(1, N_DEV - 1):
            c_up = lax.rem(my + N_DEV - 1 - s, N_DEV)
            c_dn = lax.rem(my + 1 + s, N_DEV)
            p_up = [partial_rows(c_up, 0, j) for j in (0, 1)]
            p_dn = [partial_rows(c_dn, N_HALF, j) for j in (0, 1)]
            subs = []
            for j in (0, 1):
                for r_ in hops[s - 1][j]:
                    r_.wait_recv()
                split_store(merge(s, j, comm_up, res_up) + p_up[j],
                            s, j, comm_up, res_up)
                split_store(merge(s, j, comm_dn, res_dn) + p_dn[j],
                            s, j, comm_dn, res_dn)
                sub = make_hop(s, j)
                for r_ in sub:
                    r_.start()
                subs.append(sub)
            hops.append(subs)

        p_l = [partial_rows(my, 0, j) for j in (0, 1)]
        p_r = [partial_rows(my, N_HALF, j) for j in (0, 1)]
        local_max = jnp.float32(0.0)
        for j in (0, 1):
            for r_ in hops[N_DEV - 2][j]:
                r_.wait_recv()
            y_l = jnp.maximum(
                merge(N_DEV - 1, j, comm_up, res_up) + p_l[j], 0.0)
            y_r = jnp.maximum(
                merge(N_DEV - 1, j, comm_dn, res_dn) + p_r[j], 0.0)
            out_ref[rows(j), 0:N_HALF] = y_l
            out_ref[rows(j), N_HALF:N_OUT] = y_r
            local_max = jnp.maximum(
                local_max, jnp.maximum(jnp.max(y_l), jnp.max(y_r)))

        maxbuf[7, :, :] = jnp.full((8, 128), local_max, jnp.float32)
        a2a = []
        for d in range(1, N_DEV):
            tgt = lax.rem(my + d, N_DEV)
            o = N_DEV - 1 - d
            ex = pltpu.make_async_remote_copy(
                src_ref=maxbuf.at[7], dst_ref=maxbuf.at[o],
                send_sem=a2a_send.at[o], recv_sem=a2a_recv.at[o],
                device_id=(tgt,), device_id_type=pl.DeviceIdType.MESH,
            )
            ex.start()
            a2a.append(ex)
        for hop in hops:
            for sub in hop:
                for r_ in sub:
                    r_.wait_send()
        for ex in a2a:
            ex.wait_recv()
        for ex in a2a:
            ex.wait_send()

        amax = jnp.max(maxbuf[:, :, :])
        scale = amax / 448.0
        inv = jnp.where(scale > 0.0, 1.0 / scale, 0.0)
        q = (out_ref[:, :] * inv).astype(jnp.float8_e4m3fn)
        out_ref[:, :] = q.astype(jnp.float32) * scale

    return pl.pallas_call(
        body,
        out_shape=jax.ShapeDtypeStruct((M_PER, N_OUT), jnp.float32),
        in_specs=[
            pl.BlockSpec(memory_space=pltpu.VMEM),
            pl.BlockSpec(memory_space=pltpu.VMEM),
        ],
        out_specs=pl.BlockSpec(memory_space=pltpu.VMEM),
        scratch_shapes=[
            pltpu.VMEM((N_DEV, M_PER, N_HALF), jnp.bfloat16),
            pltpu.VMEM((N_DEV, M_PER, N_HALF), jnp.bfloat16),
            pltpu.VMEM((N_DEV, M_PER, N_HALF), jnp.float8_e5m2),
            pltpu.VMEM((N_DEV, M_PER, N_HALF), jnp.float8_e5m2),
            pltpu.VMEM((N_DEV, 8, 128), jnp.float32),
            pltpu.SemaphoreType.DMA((N_DEV - 1, 2)),
            pltpu.SemaphoreType.DMA((N_DEV - 1, 2)),
            pltpu.SemaphoreType.DMA((N_DEV - 1, 2)),
            pltpu.SemaphoreType.DMA((N_DEV - 1, 2)),
            pltpu.SemaphoreType.DMA((N_DEV - 1, 2)),
            pltpu.SemaphoreType.DMA((N_DEV - 1, 2)),
            pltpu.SemaphoreType.DMA((N_DEV - 1, 2)),
            pltpu.SemaphoreType.DMA((N_DEV - 1, 2)),
            pltpu.SemaphoreType.DMA((N_DEV - 1,)),
            pltpu.SemaphoreType.DMA((N_DEV - 1,)),
        ],
        compiler_params=pltpu.CompilerParams(
            collective_id=0, vmem_limit_bytes=100 * 1024 * 1024),
    )(x, w_mat)


# device time: 122885 ns/iter; 2.0772x vs baseline; 1.0440x over previous
import jax
import jax.numpy as jnp
from jax import lax
from jax.experimental import pallas as pl
from jax.experimental.pallas import tpu as pltpu

N_DEV = 8
M_PER = 512
N_OUT = 2048
N_HALF = N_OUT // 2
K_PER = 512


def kernel(x, w_mat):
    m_total, k_per = x.shape
    _, n_out = w_mat.shape
    assert m_total == N_DEV * M_PER and k_per == K_PER and n_out == N_OUT

    M_SUB = M_PER // 2

    def body(x_ref, w_ref, out_ref,
             comm_up, comm_dn, res_up, res_dn, maxbuf,
             send_up, recv_up, send_dn, recv_dn,
             rsend_up, rrecv_up, rsend_dn, rrecv_dn,
             a2a_send, a2a_recv):
        my = lax.axis_index("i")
        right = lax.rem(my + 1, N_DEV)
        left = lax.rem(my + N_DEV - 1, N_DEV)

        barrier_sem = pltpu.get_barrier_semaphore()
        for nbr in (left, right):
            pl.semaphore_signal(
                barrier_sem, inc=1,
                device_id=(nbr,), device_id_type=pl.DeviceIdType.MESH,
            )
        pl.semaphore_wait(barrier_sem, 2)

        def partial_rows(c, lo, j):
            xc = x_ref[pl.ds(c * M_PER + j * M_SUB, M_SUB), :]
            return lax.dot_general(
                xc, w_ref[:, lo:lo + N_HALF],
                (((1,), (0,)), ((), ())),
                precision=lax.Precision.HIGHEST,
                preferred_element_type=jnp.float32,
            )

        RES_FROM = 4

        def rows(j):
            return pl.ds(j * M_SUB, M_SUB)

        def split_store(a, s, j, comm, res):
            b = a.astype(jnp.bfloat16)
            comm[s, rows(j), :] = b
            if s >= RES_FROM:
                res[s, rows(j), :] = (a - b.astype(jnp.float32)).astype(
                    jnp.float8_e5m2)

        def merge(s, j, comm, res):
            m = comm[s, rows(j), :].astype(jnp.float32)
            if s >= RES_FROM + 1:
                m = m + res[s, rows(j), :].astype(jnp.float32)
            return m

        def make_hop(s, j):
            rdmas = []
            for comm, res, sems, tgt in (
                (comm_up, res_up, (send_up, recv_up, rsend_up, rrecv_up),
                 right),
                (comm_dn, res_dn, (send_dn, recv_dn, rsend_dn, rrecv_dn),
                 left),
            ):
                rdmas.append(pltpu.make_async_remote_copy(
                    src_ref=comm.at[s, rows(j), :],
                    dst_ref=comm.at[s + 1, rows(j), :],
                    send_sem=sems[0].at[s, j], recv_sem=sems[1].at[s, j],
                    device_id=(tgt,), device_id_type=pl.DeviceIdType.MESH,
                ))
                if s >= RES_FROM:
                    rdmas.append(pltpu.make_async_remote_copy(
                        src_ref=res.at[s, rows(j), :],
                        dst_ref=res.at[s + 1, rows(j), :],
                        send_sem=sems[2].at[s, j],
                        recv_sem=sems[3].at[s, j],
                        device_id=(tgt,),
                        device_id_type=pl.DeviceIdType.MESH,
                    ))
            return rdmas

        hops = []
        c_up0 = lax.rem(my + N_DEV - 1, N_DEV)
        c_dn0 = lax.rem(my + 1, N_DEV)
        subs = []
        for j in (0, 1):
            split_store(partial_rows(c_up0, 0, j), 0, j, comm_up, res_up)
            split_store(partial_rows(c_dn0, N_HALF, j), 0, j,
                        comm_dn, res_dn)
            sub = make_hop(0, j)
            for r_ in sub:
                r_.start()
            subs.append(sub)
        hops.append(subs)

        for s in range(1, N_DEV - 1):
            c_up = lax.rem(my + N_DEV - 1 - s, N_DEV)
            c_dn = lax.rem(my + 1 + s, N_DEV)
            p_up = [partial_rows(c_up, 0, j) for j in (0, 1)]
            p_dn = [partial_rows(c_dn, N_HALF, j) for j in (0, 1)]
            subs = []
            for j in (0, 1):
                for r_ in hops[s - 1][j]:
                    r_.wait_recv()
                split_store(merge(s, j, comm_up, res_up) + p_up[j],
                            s, j, comm_up, res_up)
                split_store(merge(s, j, comm_dn, res_dn) + p_dn[j],
                            s, j, comm_dn, res_dn)
                sub = make_hop(s, j)
                for r_ in sub:
                    r_.start()
                subs.append(sub)
            hops.append(subs)

        p_l = [partial_rows(my, 0, j) for j in (0, 1)]
        p_r = [partial_rows(my, N_HALF, j) for j in (0, 1)]
        local_max = jnp.float32(0.0)
        for j in (0, 1):
            for r_ in hops[N_DEV - 2][j]:
                r_.wait_recv()
            y_l = jnp.maximum(
                merge(N_DEV - 1, j, comm_up, res_up) + p_l[j], 0.0)
            y_r = jnp.maximum(
                merge(N_DEV - 1, j, comm_dn, res_dn) + p_r[j], 0.0)
            out_ref[rows(j), 0:N_HALF] = y_l
            out_ref[rows(j), N_HALF:N_OUT] = y_r
            local_max = jnp.maximum(
                local_max, jnp.maximum(jnp.max(y_l), jnp.max(y_r)))

        maxbuf[7, :, :] = jnp.full((8, 128), local_max, jnp.float32)
        a2a = []
        for d in range(1, N_DEV):
            tgt = lax.rem(my + d, N_DEV)
            o = N_DEV - 1 - d
            ex = pltpu.make_async_remote_copy(
                src_ref=maxbuf.at[7], dst_ref=maxbuf.at[o],
                send_sem=a2a_send.at[o], recv_sem=a2a_recv.at[o],
                device_id=(tgt,), device_id_type=pl.DeviceIdType.MESH,
            )
            ex.start()
            a2a.append(ex)
        for hop in hops:
            for sub in hop:
                for r_ in sub:
                    r_.wait_send()
        for ex in a2a:
            ex.wait_recv()
        for ex in a2a:
            ex.wait_send()

        amax = jnp.max(maxbuf[:, :, :])
        scale = amax / 448.0
        inv = jnp.where(scale > 0.0, 1.0 / scale, 0.0)
        q = (out_ref[:, :] * inv).astype(jnp.float8_e4m3fn)
        out_ref[:, :] = q.astype(jnp.float32) * scale

    return pl.pallas_call(
        body,
        out_shape=jax.ShapeDtypeStruct((M_PER, N_OUT), jnp.float32),
        in_specs=[
            pl.BlockSpec(memory_space=pltpu.VMEM),
            pl.BlockSpec(memory_space=pltpu.VMEM),
        ],
        out_specs=pl.BlockSpec(memory_space=pltpu.VMEM),
        scratch_shapes=[
            pltpu.VMEM((N_DEV, M_PER, N_HALF), jnp.bfloat16),
            pltpu.VMEM((N_DEV, M_PER, N_HALF), jnp.bfloat16),
            pltpu.VMEM((N_DEV, M_PER, N_HALF), jnp.float8_e5m2),
            pltpu.VMEM((N_DEV, M_PER, N_HALF), jnp.float8_e5m2),
            pltpu.VMEM((N_DEV, 8, 128), jnp.float32),
            pltpu.SemaphoreType.DMA((N_DEV - 1, 2)),
            pltpu.SemaphoreType.DMA((N_DEV - 1, 2)),
            pltpu.SemaphoreType.DMA((N_DEV - 1, 2)),
            pltpu.SemaphoreType.DMA((N_DEV - 1, 2)),
            pltpu.SemaphoreType.DMA((N_DEV - 1, 2)),
            pltpu.SemaphoreType.DMA((N_DEV - 1, 2)),
            pltpu.SemaphoreType.DMA((N_DEV - 1, 2)),
            pltpu.SemaphoreType.DMA((N_DEV - 1, 2)),
            pltpu.SemaphoreType.DMA((N_DEV - 1,)),
            pltpu.SemaphoreType.DMA((N_DEV - 1,)),
        ],
        compiler_params=pltpu.CompilerParams(
            collective_id=0, vmem_limit_bytes=100 * 1024 * 1024),
    )(x, w_mat)


# device time: 107880 ns/iter; 2.3662x vs baseline; 1.1391x over previous
import jax
import jax.numpy as jnp
from jax import lax
from jax.experimental import pallas as pl
from jax.experimental.pallas import tpu as pltpu

N_DEV = 8
M_PER = 512
N_OUT = 2048
K_PER = 512

MX, MY, MZ = 1, 3, 4
SCHEDS = (
    (MX, MY, MZ, 0, 768),
    (MY, MZ, MX, 768, 1408),
    (MZ, MX, MY, 1408, 2048),
)


def kernel(x, w_mat):
    m_total, k_per = x.shape
    _, n_out = w_mat.shape
    assert m_total == N_DEV * M_PER and k_per == K_PER and n_out == N_OUT

    def body(x_ref, w_ref, out_ref,
             sb0, rb0, sb1, rb1, sb2, rb2, maxbuf,
             ssend, srecv, a2a_send, a2a_recv):
        my = lax.axis_index("i")
        sbufs = (sb0, sb1, sb2)
        rbufs = (rb0, rb1, rb2)

        barrier_sem = pltpu.get_barrier_semaphore()
        for m in (MX, MY, MZ):
            pl.semaphore_signal(
                barrier_sem, inc=1,
                device_id=(jnp.bitwise_xor(my, m),),
                device_id_type=pl.DeviceIdType.MESH,
            )
        pl.semaphore_wait(barrier_sem, 3)

        def partial(bmask, i):
            b = jnp.bitwise_xor(my, bmask)
            lo, hi = SCHEDS[i][3], SCHEDS[i][4]
            xc = x_ref[pl.ds(b * M_PER, M_PER), :]
            return lax.dot_general(
                xc, w_ref[:, lo:hi],
                (((1,), (0,)), ((), ())),
                precision=lax.Precision.HIGHEST,
                preferred_element_type=jnp.float32,
            )

        def mk(i, slot, tgt_mask):
            return pltpu.make_async_remote_copy(
                src_ref=sbufs[i].at[slot], dst_ref=rbufs[i].at[slot],
                send_sem=ssend.at[i, slot], recv_sem=srecv.at[i, slot],
                device_id=(jnp.bitwise_xor(my, tgt_mask),),
                device_id_type=pl.DeviceIdType.MESH,
            )

        diff1 = []
        for (m1, m2, m3, _, _) in SCHEDS:
            diff1.append((m1 ^ m2, m1 ^ m2 ^ m3, m1 ^ m3, m1))

        rdmas = []

        st1 = [[None] * 4 for _ in range(3)]
        for k in range(4):
            for i, (m1, m2, m3, lo, hi) in enumerate(SCHEDS):
                sbufs[i][k, :, :] = partial(diff1[i][k], i).astype(
                    jnp.bfloat16)
                r = mk(i, k, m1)
                r.start()
                st1[i][k] = r
                rdmas.append(r)

        st2 = [[None] * 2 for _ in range(3)]
        for k2 in range(2):
            for i, (m1, m2, m3, lo, hi) in enumerate(SCHEDS):
                bm = m2 if k2 == 0 else (m2 ^ m3)
                p = partial(bm, i)
                st1[i][k2].wait_recv()
                acc = rbufs[i][k2, :, :].astype(jnp.float32) + p
                sbufs[i][4 + k2, :, :] = acc.astype(jnp.bfloat16)
                r = mk(i, 4 + k2, m2)
                r.start()
                st2[i][k2] = r
                rdmas.append(r)

        st3 = [None] * 3
        for i, (m1, m2, m3, lo, hi) in enumerate(SCHEDS):
            p = partial(m3, i)
            st2[i][1].wait_recv()
            st1[i][2].wait_recv()
            acc = (rbufs[i][5, :, :].astype(jnp.float32)
                   + rbufs[i][2, :, :].astype(jnp.float32) + p)
            sbufs[i][6, :, :] = acc.astype(jnp.bfloat16)
            r = mk(i, 6, m3)
            r.start()
            st3[i] = r
            rdmas.append(r)

        local_max = jnp.float32(0.0)
        for i, (m1, m2, m3, lo, hi) in enumerate(SCHEDS):
            p = partial(0, i)
            st3[i].wait_recv()
            st2[i][0].wait_recv()
            st1[i][3].wait_recv()
            y = (rbufs[i][6, :, :].astype(jnp.float32)
                 + rbufs[i][4, :, :].astype(jnp.float32)
                 + rbufs[i][3, :, :].astype(jnp.float32) + p)
            y = jnp.maximum(y, 0.0)
            out_ref[:, lo:hi] = y
            local_max = jnp.maximum(local_max, jnp.max(y))

        maxbuf[7, :, :] = jnp.full((8, 128), local_max, jnp.float32)
        a2a = []
        for d in range(1, N_DEV):
            tgt = lax.rem(my + d, N_DEV)
            o = N_DEV - 1 - d
            ex = pltpu.make_async_remote_copy(
                src_ref=maxbuf.at[7], dst_ref=maxbuf.at[o],
                send_sem=a2a_send.at[o], recv_sem=a2a_recv.at[o],
                device_id=(tgt,), device_id_type=pl.DeviceIdType.MESH,
            )
            ex.start()
            a2a.append(ex)
        for r in rdmas:
            r.wait_send()
        for ex in a2a:
            ex.wait_recv()
        for ex in a2a:
            ex.wait_send()

        amax = jnp.max(maxbuf[:, :, :])
        scale = amax / 448.0
        inv = jnp.where(scale > 0.0, 1.0 / scale, 0.0)
        q = (out_ref[:, :] * inv).astype(jnp.float8_e4m3fn)
        out_ref[:, :] = q.astype(jnp.float32) * scale

    return pl.pallas_call(
        body,
        out_shape=jax.ShapeDtypeStruct((M_PER, N_OUT), jnp.float32),
        in_specs=[
            pl.BlockSpec(memory_space=pltpu.VMEM),
            pl.BlockSpec(memory_space=pltpu.VMEM),
        ],
        out_specs=pl.BlockSpec(memory_space=pltpu.VMEM),
        scratch_shapes=[
            pltpu.VMEM((7, M_PER, 768), jnp.bfloat16),
            pltpu.VMEM((7, M_PER, 768), jnp.bfloat16),
            pltpu.VMEM((7, M_PER, 640), jnp.bfloat16),
            pltpu.VMEM((7, M_PER, 640), jnp.bfloat16),
            pltpu.VMEM((7, M_PER, 640), jnp.bfloat16),
            pltpu.VMEM((7, M_PER, 640), jnp.bfloat16),
            pltpu.VMEM((N_DEV, 8, 128), jnp.float32),
            pltpu.SemaphoreType.DMA((3, 7)),
            pltpu.SemaphoreType.DMA((3, 7)),
            pltpu.SemaphoreType.DMA((N_DEV - 1,)),
            pltpu.SemaphoreType.DMA((N_DEV - 1,)),
        ],
        compiler_params=pltpu.CompilerParams(
            collective_id=0, vmem_limit_bytes=100 * 1024 * 1024),
    )(x, w_mat)


# device time: 94093 ns/iter; 2.7129x vs baseline; 1.1465x over previous
import jax
import jax.numpy as jnp
from jax import lax
from jax.experimental import pallas as pl
from jax.experimental.pallas import tpu as pltpu

N_DEV = 8
M_PER = 512
N_OUT = 2048
K_PER = 512

MX, MY, MZ = 1, 3, 4
SCHEDS = (
    (MX, MY, MZ, 0, 768),
    (MY, MZ, MX, 768, 1408),
    (MZ, MX, MY, 1408, 2048),
)


def kernel(x, w_mat):
    m_total, k_per = x.shape
    _, n_out = w_mat.shape
    assert m_total == N_DEV * M_PER and k_per == K_PER and n_out == N_OUT

    def body(x_ref, w_ref, out_ref,
             sb0, rb0, sb1, rb1, sb2, rb2,
             qs0, qr0, qs1, qr1, qs2, qr2, maxbuf,
             ssend, srecv, qsend, qrecv, a2a_send, a2a_recv):
        my = lax.axis_index("i")
        sbufs = (sb0, sb1, sb2)
        rbufs = (rb0, rb1, rb2)
        qsbufs = (qs0, qs1, qs2)
        qrbufs = (qr0, qr1, qr2)

        barrier_sem = pltpu.get_barrier_semaphore()
        for m in (MX, MY, MZ):
            pl.semaphore_signal(
                barrier_sem, inc=1,
                device_id=(jnp.bitwise_xor(my, m),),
                device_id_type=pl.DeviceIdType.MESH,
            )
        pl.semaphore_wait(barrier_sem, 3)

        def partial(bmask, i):
            b = jnp.bitwise_xor(my, bmask)
            lo, hi = SCHEDS[i][3], SCHEDS[i][4]
            xc = x_ref[pl.ds(b * M_PER, M_PER), :]
            return lax.dot_general(
                xc, w_ref[:, lo:hi],
                (((1,), (0,)), ((), ())),
                precision=lax.Precision.DEFAULT,
                preferred_element_type=jnp.float32,
            )

        def mk(i, slot, tgt_mask):
            return pltpu.make_async_remote_copy(
                src_ref=sbufs[i].at[slot], dst_ref=rbufs[i].at[slot],
                send_sem=ssend.at[i, slot], recv_sem=srecv.at[i, slot],
                device_id=(jnp.bitwise_xor(my, tgt_mask),),
                device_id_type=pl.DeviceIdType.MESH,
            )

        def mkq(i, qslot, tgt_mask):
            return pltpu.make_async_remote_copy(
                src_ref=qsbufs[i].at[qslot], dst_ref=qrbufs[i].at[qslot],
                send_sem=qsend.at[i, qslot], recv_sem=qrecv.at[i, qslot],
                device_id=(jnp.bitwise_xor(my, tgt_mask),),
                device_id_type=pl.DeviceIdType.MESH,
            )

        def split_store(acc, i, slot, qslot):
            b = acc.astype(jnp.bfloat16)
            sbufs[i][slot, :, :] = b
            qsbufs[i][qslot, :, :] = (
                acc - b.astype(jnp.float32)).astype(jnp.float8_e5m2)

        diff1 = []
        for (m1, m2, m3, _, _) in SCHEDS:
            diff1.append((m1 ^ m2, m1 ^ m2 ^ m3, m1 ^ m3, m1))

        rdmas = []

        st1 = [[None] * 4 for _ in range(3)]
        for k in range(4):
            for i, (m1, m2, m3, lo, hi) in enumerate(SCHEDS):
                sbufs[i][k, :, :] = partial(diff1[i][k], i).astype(
                    jnp.bfloat16)
                r = mk(i, k, m1)
                r.start()
                st1[i][k] = r
                rdmas.append(r)

        st2 = [[None] * 2 for _ in range(3)]
        st2q = [[None] * 2 for _ in range(3)]
        for k2 in range(2):
            for i, (m1, m2, m3, lo, hi) in enumerate(SCHEDS):
                bm = m2 if k2 == 0 else (m2 ^ m3)
                p = partial(bm, i)
                st1[i][k2].wait_recv()
                acc = rbufs[i][k2, :, :].astype(jnp.float32) + p
                split_store(acc, i, 4 + k2, k2)
                r = mk(i, 4 + k2, m2)
                rq = mkq(i, k2, m2)
                r.start()
                rq.start()
                st2[i][k2] = r
                st2q[i][k2] = rq
                rdmas.extend((r, rq))

        st3 = [None] * 3
        st3q = [None] * 3
        for i, (m1, m2, m3, lo, hi) in enumerate(SCHEDS):
            p = partial(m3, i)
            st2[i][1].wait_recv()
            st2q[i][1].wait_recv()
            st1[i][2].wait_recv()
            acc = (rbufs[i][5, :, :].astype(jnp.float32)
                   + qrbufs[i][1, :, :].astype(jnp.float32)
                   + rbufs[i][2, :, :].astype(jnp.float32) + p)
            split_store(acc, i, 6, 2)
            r = mk(i, 6, m3)
            rq = mkq(i, 2, m3)
            r.start()
            rq.start()
            st3[i] = r
            st3q[i] = rq
            rdmas.extend((r, rq))

        local_max = jnp.float32(0.0)
        for i, (m1, m2, m3, lo, hi) in enumerate(SCHEDS):
            p = partial(0, i)
            st3[i].wait_recv()
            st3q[i].wait_recv()
            st2[i][0].wait_recv()
            st2q[i][0].wait_recv()
            st1[i][3].wait_recv()
            y = (rbufs[i][6, :, :].astype(jnp.float32)
                 + qrbufs[i][2, :, :].astype(jnp.float32)
                 + rbufs[i][4, :, :].astype(jnp.float32)
                 + qrbufs[i][0, :, :].astype(jnp.float32)
                 + rbufs[i][3, :, :].astype(jnp.float32) + p)
            y = jnp.maximum(y, 0.0)
            out_ref[:, lo:hi] = y
            local_max = jnp.maximum(local_max, jnp.max(y))

        maxbuf[7, :, :] = jnp.full((8, 128), local_max, jnp.float32)
        a2a = []
        for d in range(1, N_DEV):
            tgt = lax.rem(my + d, N_DEV)
            o = N_DEV - 1 - d
            ex = pltpu.make_async_remote_copy(
                src_ref=maxbuf.at[7], dst_ref=maxbuf.at[o],
                send_sem=a2a_send.at[o], recv_sem=a2a_recv.at[o],
                device_id=(tgt,), device_id_type=pl.DeviceIdType.MESH,
            )
            ex.start()
            a2a.append(ex)
        for r in rdmas:
            r.wait_send()
        for ex in a2a:
            ex.wait_recv()
        for ex in a2a:
            ex.wait_send()

        amax = jnp.max(maxbuf[:, :, :])
        scale = amax / 448.0
        inv = jnp.where(scale > 0.0, 1.0 / scale, 0.0)
        q = (out_ref[:, :] * inv).astype(jnp.float8_e4m3fn)
        out_ref[:, :] = q.astype(jnp.float32) * scale

    return pl.pallas_call(
        body,
        out_shape=jax.ShapeDtypeStruct((M_PER, N_OUT), jnp.float32),
        in_specs=[
            pl.BlockSpec(memory_space=pltpu.VMEM),
            pl.BlockSpec(memory_space=pltpu.VMEM),
        ],
        out_specs=pl.BlockSpec(memory_space=pltpu.VMEM),
        scratch_shapes=[
            pltpu.VMEM((7, M_PER, 768), jnp.bfloat16),
            pltpu.VMEM((7, M_PER, 768), jnp.bfloat16),
            pltpu.VMEM((7, M_PER, 640), jnp.bfloat16),
            pltpu.VMEM((7, M_PER, 640), jnp.bfloat16),
            pltpu.VMEM((7, M_PER, 640), jnp.bfloat16),
            pltpu.VMEM((7, M_PER, 640), jnp.bfloat16),
            pltpu.VMEM((3, M_PER, 768), jnp.float8_e5m2),
            pltpu.VMEM((3, M_PER, 768), jnp.float8_e5m2),
            pltpu.VMEM((3, M_PER, 640), jnp.float8_e5m2),
            pltpu.VMEM((3, M_PER, 640), jnp.float8_e5m2),
            pltpu.VMEM((3, M_PER, 640), jnp.float8_e5m2),
            pltpu.VMEM((3, M_PER, 640), jnp.float8_e5m2),
            pltpu.VMEM((N_DEV, 8, 128), jnp.float32),
            pltpu.SemaphoreType.DMA((3, 7)),
            pltpu.SemaphoreType.DMA((3, 7)),
            pltpu.SemaphoreType.DMA((3, 3)),
            pltpu.SemaphoreType.DMA((3, 3)),
            pltpu.SemaphoreType.DMA((N_DEV - 1,)),
            pltpu.SemaphoreType.DMA((N_DEV - 1,)),
        ],
        compiler_params=pltpu.CompilerParams(
            collective_id=0, vmem_limit_bytes=100 * 1024 * 1024),
    )(x, w_mat)


# device time: 89042 ns/iter; 2.8667x vs baseline; 1.0567x over previous
import jax
import jax.numpy as jnp
from jax import lax
from jax.experimental import pallas as pl
from jax.experimental.pallas import tpu as pltpu

N_DEV = 8
M_PER = 512
N_OUT = 2048
K_PER = 512

MX, MY, MZ = 1, 3, 4
SCHEDS = (
    (MX, MY, MZ, 0, 768),
    (MY, MZ, MX, 768, 1408),
    (MZ, MX, MY, 1408, 2048),
)


def kernel(x, w_mat):
    m_total, k_per = x.shape
    _, n_out = w_mat.shape
    assert m_total == N_DEV * M_PER and k_per == K_PER and n_out == N_OUT

    def body(x_ref, w_ref, out_ref,
             sb0, rb0, sb1, rb1, sb2, rb2,
             qs0, qr0, qs1, qr1, qs2, qr2, maxbuf,
             ssend, srecv, qsend, qrecv, a2a_send, a2a_recv):
        my = lax.axis_index("i")
        sbufs = (sb0, sb1, sb2)
        rbufs = (rb0, rb1, rb2)
        qsbufs = (qs0, qs1, qs2)
        qrbufs = (qr0, qr1, qr2)

        barrier_sem = pltpu.get_barrier_semaphore()
        for m in (MX, MY, MZ):
            pl.semaphore_signal(
                barrier_sem, inc=1,
                device_id=(jnp.bitwise_xor(my, m),),
                device_id_type=pl.DeviceIdType.MESH,
            )
        pl.semaphore_wait(barrier_sem, 3)

        def partial(bmask, i):
            b = jnp.bitwise_xor(my, bmask)
            lo, hi = SCHEDS[i][3], SCHEDS[i][4]
            xc = x_ref[pl.ds(b * M_PER, M_PER), :]
            return lax.dot_general(
                xc, w_ref[:, lo:hi],
                (((1,), (0,)), ((), ())),
                precision=lax.Precision.DEFAULT,
                preferred_element_type=jnp.float32,
            )

        def mk(i, slot, tgt_mask):
            return pltpu.make_async_remote_copy(
                src_ref=sbufs[i].at[slot], dst_ref=rbufs[i].at[slot],
                send_sem=ssend.at[i, slot], recv_sem=srecv.at[i, slot],
                device_id=(jnp.bitwise_xor(my, tgt_mask),),
                device_id_type=pl.DeviceIdType.MESH,
            )

        def mkq(i, qslot, tgt_mask):
            return pltpu.make_async_remote_copy(
                src_ref=qsbufs[i].at[qslot], dst_ref=qrbufs[i].at[qslot],
                send_sem=qsend.at[i, qslot], recv_sem=qrecv.at[i, qslot],
                device_id=(jnp.bitwise_xor(my, tgt_mask),),
                device_id_type=pl.DeviceIdType.MESH,
            )

        def split_store(acc, i, slot, qslot):
            b = acc.astype(jnp.bfloat16)
            sbufs[i][slot, :, :] = b
            qsbufs[i][qslot, :, :] = (
                acc - b.astype(jnp.float32)).astype(jnp.float8_e5m2)

        diff1 = []
        for (m1, m2, m3, _, _) in SCHEDS:
            diff1.append((m1 ^ m2 ^ m3, m1 ^ m2, m1 ^ m3, m1))

        rdmas = []

        st1 = [[None] * 4 for _ in range(3)]
        for k in range(4):
            for i, (m1, m2, m3, lo, hi) in enumerate(SCHEDS):
                sbufs[i][k, :, :] = partial(diff1[i][k], i).astype(
                    jnp.bfloat16)
                r = mk(i, k, m1)
                r.start()
                st1[i][k] = r
                rdmas.append(r)

        st2 = [[None] * 2 for _ in range(3)]
        st2q = [[None] * 2 for _ in range(3)]
        for k2 in range(2):
            for i, (m1, m2, m3, lo, hi) in enumerate(SCHEDS):
                bm = (m2 ^ m3) if k2 == 0 else m2
                p = partial(bm, i)
                st1[i][k2].wait_recv()
                acc = rbufs[i][k2, :, :].astype(jnp.float32) + p
                split_store(acc, i, 4 + k2, k2)
                r = mk(i, 4 + k2, m2)
                rq = mkq(i, k2, m2)
                r.start()
                rq.start()
                st2[i][k2] = r
                st2q[i][k2] = rq
                rdmas.extend((r, rq))

        st3 = [None] * 3
        st3q = [None] * 3
        for i, (m1, m2, m3, lo, hi) in enumerate(SCHEDS):
            p = partial(m3, i)
            st2[i][0].wait_recv()
            st2q[i][0].wait_recv()
            st1[i][2].wait_recv()
            acc = (rbufs[i][4, :, :].astype(jnp.float32)
                   + qrbufs[i][0, :, :].astype(jnp.float32)
                   + rbufs[i][2, :, :].astype(jnp.float32) + p)
            split_store(acc, i, 6, 2)
            r = mk(i, 6, m3)
            rq = mkq(i, 2, m3)
            r.start()
            rq.start()
            st3[i] = r
            st3q[i] = rq
            rdmas.extend((r, rq))

        local_max = jnp.float32(0.0)
        for i, (m1, m2, m3, lo, hi) in enumerate(SCHEDS):
            p = partial(0, i)
            st3[i].wait_recv()
            st3q[i].wait_recv()
            st2[i][1].wait_recv()
            st2q[i][1].wait_recv()
            st1[i][3].wait_recv()
            y = (rbufs[i][6, :, :].astype(jnp.float32)
                 + qrbufs[i][2, :, :].astype(jnp.float32)
                 + rbufs[i][5, :, :].astype(jnp.float32)
                 + qrbufs[i][1, :, :].astype(jnp.float32)
                 + rbufs[i][3, :, :].astype(jnp.float32) + p)
            y = jnp.maximum(y, 0.0)
            out_ref[:, lo:hi] = y
            local_max = jnp.maximum(local_max, jnp.max(y))

        maxbuf[7, :, :] = jnp.full((8, 128), local_max, jnp.float32)
        a2a = []
        for d in range(1, N_DEV):
            tgt = lax.rem(my + d, N_DEV)
            o = N_DEV - 1 - d
            ex = pltpu.make_async_remote_copy(
                src_ref=maxbuf.at[7], dst_ref=maxbuf.at[o],
                send_sem=a2a_send.at[o], recv_sem=a2a_recv.at[o],
                device_id=(tgt,), device_id_type=pl.DeviceIdType.MESH,
            )
            ex.start()
            a2a.append(ex)
        for r in rdmas:
            r.wait_send()
        for ex in a2a:
            ex.wait_recv()
        for ex in a2a:
            ex.wait_send()

        amax = jnp.max(maxbuf[:, :, :])
        scale = amax / 448.0
        inv = jnp.where(scale > 0.0, 1.0 / scale, 0.0)
        q = (out_ref[:, :] * inv).astype(jnp.float8_e4m3fn)
        out_ref[:, :] = q.astype(jnp.float32) * scale

    return pl.pallas_call(
        body,
        out_shape=jax.ShapeDtypeStruct((M_PER, N_OUT), jnp.float32),
        in_specs=[
            pl.BlockSpec(memory_space=pltpu.VMEM),
            pl.BlockSpec(memory_space=pltpu.VMEM),
        ],
        out_specs=pl.BlockSpec(memory_space=pltpu.VMEM),
        scratch_shapes=[
            pltpu.VMEM((7, M_PER, 768), jnp.bfloat16),
            pltpu.VMEM((7, M_PER, 768), jnp.bfloat16),
            pltpu.VMEM((7, M_PER, 640), jnp.bfloat16),
            pltpu.VMEM((7, M_PER, 640), jnp.bfloat16),
            pltpu.VMEM((7, M_PER, 640), jnp.bfloat16),
            pltpu.VMEM((7, M_PER, 640), jnp.bfloat16),
            pltpu.VMEM((3, M_PER, 768), jnp.float8_e5m2),
            pltpu.VMEM((3, M_PER, 768), jnp.float8_e5m2),
            pltpu.VMEM((3, M_PER, 640), jnp.float8_e5m2),
            pltpu.VMEM((3, M_PER, 640), jnp.float8_e5m2),
            pltpu.VMEM((3, M_PER, 640), jnp.float8_e5m2),
            pltpu.VMEM((3, M_PER, 640), jnp.float8_e5m2),
            pltpu.VMEM((N_DEV, 8, 128), jnp.float32),
            pltpu.SemaphoreType.DMA((3, 7)),
            pltpu.SemaphoreType.DMA((3, 7)),
            pltpu.SemaphoreType.DMA((3, 3)),
            pltpu.SemaphoreType.DMA((3, 3)),
            pltpu.SemaphoreType.DMA((N_DEV - 1,)),
            pltpu.SemaphoreType.DMA((N_DEV - 1,)),
        ],
        compiler_params=pltpu.CompilerParams(
            collective_id=0, vmem_limit_bytes=100 * 1024 * 1024),
    )(x, w_mat)
